# baseline (speedup 1.0000x reference)
import jax
import jax.numpy as jnp
from jax import lax
from jax.experimental import pallas as pl
from jax.experimental.pallas import tpu as pltpu
from jax.experimental.pallas import tpu_sc as plsc

NC, NS, L = 2, 16, 16
NW = NC * NS

D = 64
DP = 128
IW = 128
R = 4
NBUF = 2
CI = 256


def _transpose_body(tt_hbm, aux_hbm, tp_hbm, in_v, out_v,
                    isem0, isem1, osem0, osem1):
    V = tt_hbm.shape[1]
    n_full = V // CI
    tail = V - n_full * CI
    wid = lax.axis_index("s") * NC + lax.axis_index("c")
    isems = (isem0, isem1)
    osems = (osem0, osem1)

    def in_copy(s, b):
        return pltpu.make_async_copy(tt_hbm.at[:, pl.ds(s * CI, CI)],
                                     in_v.at[b], isems[b])

    def out_copy(s, b):
        return pltpu.make_async_copy(out_v.at[b],
                                     tp_hbm.at[pl.ds(s * CI, CI)], osems[b])

    def transpose_slab(b):
        def col(d, _):
            for g in range(CI // L):
                vals = in_v[b, d, pl.ds(g * L, L)]
                rows = g * L + lax.iota(jnp.int32, L)
                cols = jnp.full((L,), 0, jnp.int32) + d
                plsc.store_scatter(out_v.at[b], [rows, cols], vals)
            return ()
        lax.fori_loop(0, D, col, (), unroll=False)

    if tail:
        @pl.when(wid == 0)
        def _():
            pltpu.sync_copy(aux_hbm, in_v.at[0, :, pl.ds(0, DP)])
            pltpu.sync_copy(in_v.at[0, pl.ds(0, tail), pl.ds(0, DP)],
                            tp_hbm.at[pl.ds(n_full * CI, tail)])

    @pl.when(wid < n_full)
    def _():
        in_copy(wid, 0).start()

    def step(t, _):
        for b in range(2):
            k = 2 * t + b
            s = wid + k * NW

            @pl.when(s < n_full)
            def _():
                @pl.when(s + NW < n_full)
                def _():
                    in_copy(s + NW, 1 - b).start()

                in_copy(s, b).wait()

                @pl.when(k >= 2)
                def _():
                    out_copy(s - 2 * NW, b).wait()

                transpose_slab(b)
                out_copy(s, b).start()

        return ()

    max_k = (n_full + NW - 1) // NW
    n_loop = (max_k + 1) // 2
    lax.fori_loop(0, n_loop, step, (), unroll=False)
    for b in range(2):
        @pl.when(_last_slab(n_full, wid, b) >= 0)
        def _():
            out_copy(_last_slab(n_full, wid, b), b).wait()


def _last_slab(n_full, wid, b):
    k_count = (n_full - 1 - wid) // NW + 1
    k_count = jnp.maximum(k_count, 0)
    last_k = k_count - 1
    lk = jnp.where(lax.rem(last_k, 2) == b, last_k, last_k - 1)
    return jnp.where(lk >= 0, wid + lk * NW, -1)


def _gather_body(table_hbm, idx_hbm, out_hbm, idx_v, rows_v, sem0, sem1):
    n_rows = idx_hbm.shape[0]
    rows_per_w = n_rows // NW
    n_chunks = rows_per_w // R
    wid = lax.axis_index("s") * NC + lax.axis_index("c")
    base = wid * rows_per_w
    sems = (sem0, sem1)

    def stage(g, b):
        row0 = base + g * R
        pltpu.sync_copy(idx_hbm.at[pl.ds(row0, R)], idx_v.at[b])
        for j in range(R):
            pltpu.async_copy(table_hbm.at[idx_v.at[b, j]], rows_v.at[b, j],
                             sems[b])

    stage(0, 0)

    def pair(t, _):
        for b in range(NBUF):
            g = NBUF * t + b
            nb = 1 - b

            @pl.when(g + 1 < n_chunks)
            def _():
                stage(g + 1, nb)

            pltpu.make_async_copy(table_hbm.at[idx_v.at[b]], rows_v.at[b],
                                  sems[b]).wait()
            pltpu.sync_copy(rows_v.at[b],
                            out_hbm.at[pl.ds(base + g * R, R), :, pl.ds(0, D)])
        return ()

    lax.fori_loop(0, n_chunks // NBUF, pair, (), unroll=False)


def kernel(token_ids, table):
    B, S = token_ids.shape
    V = table.shape[0]
    n_idx = B * S
    assert n_idx % (IW * NW * R * NBUF) == 0
    n_rows = n_idx // IW
    idx2d = (token_ids * 2).reshape(n_rows, IW).astype(jnp.int32)

    mesh = plsc.VectorSubcoreMesh(core_axis_name="c", subcore_axis_name="s")

    transpose = pl.kernel(
        _transpose_body,
        out_type=jax.ShapeDtypeStruct((V, DP), jnp.float32),
        mesh=mesh,
        scratch_types=[
            pltpu.VMEM((2, D, CI), jnp.float32),
            pltpu.VMEM((2, CI, DP), jnp.float32),
            pltpu.SemaphoreType.DMA,
            pltpu.SemaphoreType.DMA,
            pltpu.SemaphoreType.DMA,
            pltpu.SemaphoreType.DMA,
        ],
        compiler_params=pltpu.CompilerParams(use_tc_tiling_on_sc=True,
                                             needs_layout_passes=False),
    )
    v_main = (V // CI) * CI
    aux = jnp.pad(table[v_main:, :], ((0, 0), (0, DP - D)))
    table_p = transpose(table.T, aux)

    gather = pl.kernel(
        _gather_body,
        out_type=jax.ShapeDtypeStruct((n_rows, IW, DP), jnp.float32),
        mesh=mesh,
        scratch_types=[
            pltpu.VMEM((NBUF, R, IW), jnp.int32),
            pltpu.VMEM((NBUF, R, IW, D), jnp.float32),
            pltpu.SemaphoreType.DMA,
            pltpu.SemaphoreType.DMA,
        ],
        compiler_params=pltpu.CompilerParams(use_tc_tiling_on_sc=False),
    )
    out = gather(table_p.reshape(2 * V, D), idx2d)
    return out[:, :, :D].reshape(B, S, D)


# speedup vs baseline: 1.7822x; 1.7822x over previous
import jax
import jax.numpy as jnp
from jax import lax
from jax.experimental import pallas as pl
from jax.experimental.pallas import tpu as pltpu
from jax.experimental.pallas import tpu_sc as plsc

NC, NS, L = 2, 16, 16
NW = NC * NS

D = 64
DP = 128
IW = 128
R = 4
NBUF = 2


def _gather_body(table_hbm, idx_hbm, out_hbm, idx_v, rows_v, sem0, sem1):
    n_rows = idx_hbm.shape[0]
    rows_per_w = n_rows // NW
    n_chunks = rows_per_w // R
    wid = lax.axis_index("s") * NC + lax.axis_index("c")
    base = wid * rows_per_w
    sems = (sem0, sem1)

    def stage(g, b):
        row0 = base + g * R
        pltpu.sync_copy(idx_hbm.at[pl.ds(row0, R)], idx_v.at[b])
        for j in range(R):
            pltpu.async_copy(table_hbm.at[idx_v.at[b, j]], rows_v.at[b, j],
                             sems[b])

    stage(0, 0)

    def pair(t, _):
        for b in range(NBUF):
            g = NBUF * t + b
            nb = 1 - b

            @pl.when(g + 1 < n_chunks)
            def _():
                stage(g + 1, nb)

            pltpu.make_async_copy(table_hbm.at[idx_v.at[b]], rows_v.at[b],
                                  sems[b]).wait()
            pltpu.sync_copy(rows_v.at[b],
                            out_hbm.at[pl.ds(base + g * R, R), :, pl.ds(0, D)])
        return ()

    lax.fori_loop(0, n_chunks // NBUF, pair, (), unroll=False)


def kernel(token_ids, table):
    B, S = token_ids.shape
    V = table.shape[0]
    n_idx = B * S
    assert n_idx % (IW * NW * R * NBUF) == 0
    n_rows = n_idx // IW
    idx2d = (token_ids * 2).reshape(n_rows, IW).astype(jnp.int32)
    table_p = jnp.pad(table, ((0, 0), (0, DP - D)))

    mesh = plsc.VectorSubcoreMesh(core_axis_name="c", subcore_axis_name="s")
    gather = pl.kernel(
        _gather_body,
        out_type=jax.ShapeDtypeStruct((n_rows, IW, DP), jnp.float32),
        mesh=mesh,
        scratch_types=[
            pltpu.VMEM((NBUF, R, IW), jnp.int32),
            pltpu.VMEM((NBUF, R, IW, D), jnp.float32),
            pltpu.SemaphoreType.DMA,
            pltpu.SemaphoreType.DMA,
        ],
        compiler_params=pltpu.CompilerParams(use_tc_tiling_on_sc=False),
    )
    out = gather(table_p.reshape(2 * V, D), idx2d)
    return out[:, :, :D].reshape(B, S, D)
